# baseline (speedup 1.0000x reference)
import functools

import jax
import jax.numpy as jnp
from jax import lax
from jax.experimental import pallas as pl
from jax.experimental.pallas import tpu as pltpu
from jax.experimental.pallas import tpu_sc as plsc

_NUM_CORES = 2
_NUM_SUBCORES = 16
_NUM_WORKERS = _NUM_CORES * _NUM_SUBCORES

_B_BLK = 64


@functools.cache
def _make_sc_gather(V, D, B):
    b_per_w = B // _NUM_WORKERS
    mesh = plsc.VectorSubcoreMesh(core_axis_name="c", subcore_axis_name="s")

    @functools.partial(
        pl.kernel,
        mesh=mesh,
        out_type=jax.ShapeDtypeStruct((B, D), jnp.float32),
        scratch_types=[
            pltpu.VMEM((b_per_w,), jnp.int32),
            pltpu.VMEM((b_per_w, D), jnp.float32),
            pltpu.SemaphoreType.DMA,
        ],
        compiler_params=pltpu.CompilerParams(use_tc_tiling_on_sc=False),
    )
    def sc_gather(table_hbm, idx_hbm, out_hbm, idx_v, rows_v, sem):
        wid = lax.axis_index("s") * _NUM_CORES + lax.axis_index("c")
        base = wid * b_per_w
        pltpu.sync_copy(idx_hbm.at[pl.ds(base, b_per_w)], idx_v)
        pltpu.async_copy(table_hbm.at[idx_v], rows_v, sem).wait()
        pltpu.sync_copy(rows_v, out_hbm.at[pl.ds(base, b_per_w)])

    return sc_gather


def _tc_matmul_body(emb_ref, wt_ref, b_ref, out_ref):
    out_ref[...] = (
        lax.dot_general(
            emb_ref[...],
            wt_ref[...],
            (((1,), (0,)), ((), ())),
            preferred_element_type=jnp.float32,
        )
        + b_ref[...]
    )


@functools.cache
def _make_tc_matmul(V, D, B):
    v_aligned = (V // 128) * 128
    nsteps = B // _B_BLK
    return pl.pallas_call(
        _tc_matmul_body,
        grid=(nsteps,),
        in_specs=[
            pl.BlockSpec((_B_BLK, D), lambda i: (i, 0)),
            pl.BlockSpec((D, v_aligned), lambda i: (0, 0)),
            pl.BlockSpec((1, v_aligned), lambda i: (0, 0)),
        ],
        out_specs=pl.BlockSpec((_B_BLK, v_aligned), lambda i: (i, 0)),
        out_shape=jax.ShapeDtypeStruct((B, V), jnp.float32),
        compiler_params=pltpu.CompilerParams(
            vmem_limit_bytes=110 * 1024 * 1024,
        ),
    )


@functools.cache
def _make_tc_tail(V, D, B):
    blk_idx = V // 128

    def body(big_ref, emb_ref, wt_ref, b_ref, out_ref):
        del big_ref
        out_ref[...] = (
            lax.dot_general(
                emb_ref[...],
                wt_ref[...],
                (((1,), (0,)), ((), ())),
                preferred_element_type=jnp.float32,
            )
            + b_ref[...]
        )

    return pl.pallas_call(
        body,
        grid=(1,),
        in_specs=[
            pl.BlockSpec(memory_space=pltpu.HBM),
            pl.BlockSpec((B, D), lambda i: (0, 0)),
            pl.BlockSpec((D, 128), lambda i: (0, 0)),
            pl.BlockSpec((1, 128), lambda i: (0, 0)),
        ],
        out_specs=pl.BlockSpec((B, 128), lambda i: (0, blk_idx)),
        out_shape=jax.ShapeDtypeStruct((B, V), jnp.float32),
        input_output_aliases={0: 0},
    )


def kernel(x, tok_embeddings, fc_w, fc_b):
    V, D = tok_embeddings.shape
    B = x.shape[0]
    emb = _make_sc_gather(V, D, B)(tok_embeddings, x.astype(jnp.int32))
    wt = fc_w.T
    b_row = fc_b.reshape(1, V)
    v_aligned = (V // 128) * 128
    out = _make_tc_matmul(V, D, B)(emb, wt[:, :v_aligned], b_row[:, :v_aligned])
    if v_aligned < V:
        rem = V - v_aligned
        wt_tail = jnp.pad(wt[:, v_aligned:], ((0, 0), (0, 128 - rem)))
        b_tail = jnp.pad(b_row[:, v_aligned:], ((0, 0), (0, 128 - rem)))
        out = _make_tc_tail(V, D, B)(out, emb, wt_tail, b_tail)
    return out


# speedup vs baseline: 1.9062x; 1.9062x over previous
import functools

import jax
import jax.numpy as jnp
from jax import lax
from jax.experimental import pallas as pl
from jax.experimental.pallas import tpu as pltpu
from jax.experimental.pallas import tpu_sc as plsc

_NUM_CORES = 2
_NUM_SUBCORES = 16
_NUM_WORKERS = _NUM_CORES * _NUM_SUBCORES

_V_BLK = 4000


@functools.cache
def _make_sc_gather(V, D, B):
    b_per_w = B // _NUM_WORKERS
    mesh = plsc.VectorSubcoreMesh(core_axis_name="c", subcore_axis_name="s")

    @functools.partial(
        pl.kernel,
        mesh=mesh,
        out_type=jax.ShapeDtypeStruct((B, D), jnp.float32),
        scratch_types=[
            pltpu.VMEM((b_per_w,), jnp.int32),
            pltpu.VMEM((b_per_w, D), jnp.float32),
            pltpu.SemaphoreType.DMA,
        ],
        compiler_params=pltpu.CompilerParams(use_tc_tiling_on_sc=False),
    )
    def sc_gather(table_hbm, idx_hbm, out_hbm, idx_v, rows_v, sem):
        wid = lax.axis_index("s") * _NUM_CORES + lax.axis_index("c")
        base = wid * b_per_w
        pltpu.sync_copy(idx_hbm.at[pl.ds(base, b_per_w)], idx_v)
        pltpu.async_copy(table_hbm.at[idx_v], rows_v, sem).wait()
        pltpu.sync_copy(rows_v, out_hbm.at[pl.ds(base, b_per_w)])

    return sc_gather


def _tc_matmul_t_body(w_ref, emb_ref, b_ref, out_ref):
    out_ref[...] = (
        lax.dot_general(
            w_ref[...],
            emb_ref[...],
            (((1,), (1,)), ((), ())),
            preferred_element_type=jnp.float32,
        )
        + b_ref[...]
    )


@functools.cache
def _make_tc_matmul_t(V, D, B):
    nsteps = pl.cdiv(V, _V_BLK)
    return pl.pallas_call(
        _tc_matmul_t_body,
        grid=(nsteps,),
        in_specs=[
            pl.BlockSpec((_V_BLK, D), lambda i: (i, 0)),
            pl.BlockSpec((B, D), lambda i: (0, 0)),
            pl.BlockSpec((_V_BLK, 1), lambda i: (i, 0)),
        ],
        out_specs=pl.BlockSpec((_V_BLK, B), lambda i: (i, 0)),
        out_shape=jax.ShapeDtypeStruct((V, B), jnp.float32),
    )


def kernel(x, tok_embeddings, fc_w, fc_b):
    V, D = tok_embeddings.shape
    B = x.shape[0]
    emb = _make_sc_gather(V, D, B)(tok_embeddings, x.astype(jnp.int32))
    out_t = _make_tc_matmul_t(V, D, B)(fc_w, emb, fc_b.reshape(V, 1))
    return out_t.T
